# baseline (device time: 45123 ns/iter reference)
import jax
import jax.numpy as jnp
from jax import lax
from jax.experimental import pallas as pl
from jax.experimental.pallas import tpu as pltpu

N_DEV = 4
NCHUNK = 4


def kernel(x, Wp):
    b, h, w, c = x.shape
    cout = Wp.shape[1]
    hw2 = h * w // 2
    c2 = 2 * c
    rows = hw2 // NCHUNK
    n_global = (h * N_DEV) * w
    xp = x.reshape(b, hw2, c2)

    def body(
        x_hbm,
        wp_ref,
        out_hbm,
        x_vmem,
        outbuf,
        wp2_ref,
        stats_ref,
        copy_sems,
        out_sems,
        send_sems,
        recv_sems,
    ):
        my = lax.axis_index("i")

        barrier_sem = pltpu.get_barrier_semaphore()
        for d in (1, 2, 3):
            pl.semaphore_signal(
                barrier_sem,
                inc=1,
                device_id=((my + d) % N_DEV,),
                device_id_type=pl.DeviceIdType.MESH,
            )

        copies = []
        for bi in range(b):
            for ci in range(NCHUNK):
                cp = pltpu.make_async_copy(
                    x_hbm.at[bi, pl.ds(ci * rows, rows), :],
                    x_vmem.at[bi, pl.ds(ci * rows, rows), :],
                    copy_sems.at[bi * NCHUNK + ci],
                )
                cp.start()
                copies.append(cp)

        wp2_ref[:, :] = jnp.zeros((c2, 2 * cout), jnp.float32)
        wp2_ref[0:c, 0:cout] = wp_ref[:, :]
        wp2_ref[c : 2 * c, cout : 2 * cout] = wp_ref[:, :]

        for bi in range(b):
            s = None
            sq = None
            for ci in range(NCHUNK):
                copies[bi * NCHUNK + ci].wait()
                xb = x_vmem[bi, pl.ds(ci * rows, rows), :]
                ps = jnp.sum(xb, axis=0, keepdims=True)
                psq = jnp.sum(xb * xb, axis=0, keepdims=True)
                s = ps if s is None else s + ps
                sq = psq if sq is None else sq + psq
            stats_ref[N_DEV - 1, 2 * bi : 2 * bi + 1, :] = s
            stats_ref[N_DEV - 1, 2 * bi + 1 : 2 * bi + 2, :] = sq

        pl.semaphore_wait(barrier_sem, N_DEV - 1)
        sends = []
        for d in (1, 2, 3):
            rdma = pltpu.make_async_remote_copy(
                src_ref=stats_ref.at[N_DEV - 1],
                dst_ref=stats_ref.at[d - 1],
                send_sem=send_sems.at[d - 1],
                recv_sem=recv_sems.at[d - 1],
                device_id=((my + d) % N_DEV,),
                device_id_type=pl.DeviceIdType.MESH,
            )
            rdma.start()
            sends.append(rdma)
        for d in (1, 2, 3):
            recv = pltpu.make_async_remote_copy(
                src_ref=stats_ref.at[N_DEV - 1],
                dst_ref=stats_ref.at[d - 1],
                send_sem=send_sems.at[d - 1],
                recv_sem=recv_sems.at[d - 1],
                device_id=((my + d) % N_DEV,),
                device_id_type=pl.DeviceIdType.MESH,
            )
            recv.wait_recv()
        for rdma in sends:
            rdma.wait_send()

        eps = 1e-5
        inv_n = 1.0 / float(n_global)
        means = []
        scales = []
        for bi in range(b):
            ssum = (
                stats_ref[0, 2 * bi : 2 * bi + 1, :]
                + stats_ref[1, 2 * bi : 2 * bi + 1, :]
                + stats_ref[2, 2 * bi : 2 * bi + 1, :]
                + stats_ref[3, 2 * bi : 2 * bi + 1, :]
            )
            ssq = (
                stats_ref[0, 2 * bi + 1 : 2 * bi + 2, :]
                + stats_ref[1, 2 * bi + 1 : 2 * bi + 2, :]
                + stats_ref[2, 2 * bi + 1 : 2 * bi + 2, :]
                + stats_ref[3, 2 * bi + 1 : 2 * bi + 2, :]
            )
            s64 = ssum[:, 0:c] + ssum[:, c:c2]
            q64 = ssq[:, 0:c] + ssq[:, c:c2]
            mean = s64 * inv_n
            var = q64 * inv_n - mean * mean
            scale = lax.rsqrt(var + eps)
            means.append(jnp.concatenate([mean, mean], axis=1))
            scales.append(jnp.concatenate([scale, scale], axis=1))

        wp2 = wp2_ref[:, :]
        out_waits = [None, None]
        k = 0
        for bi in range(b):
            for ci in range(NCHUNK):
                slot = k % 2
                if out_waits[slot] is not None:
                    out_waits[slot].wait()
                xb = x_vmem[bi, pl.ds(ci * rows, rows), :]
                hh = (xb - means[bi]) * scales[bi]
                a = hh * jax.nn.sigmoid(hh)
                outbuf[slot] = jnp.dot(
                    a, wp2, preferred_element_type=jnp.float32
                )
                cp = pltpu.make_async_copy(
                    outbuf.at[slot],
                    out_hbm.at[bi, pl.ds(ci * rows, rows), :],
                    out_sems.at[slot],
                )
                cp.start()
                out_waits[slot] = cp
                k += 1
        out_waits[0].wait()
        out_waits[1].wait()

    out = pl.pallas_call(
        body,
        out_shape=jax.ShapeDtypeStruct((b, hw2, 2 * cout), jnp.float32),
        in_specs=[
            pl.BlockSpec(memory_space=pltpu.MemorySpace.HBM),
            pl.BlockSpec(memory_space=pltpu.MemorySpace.VMEM),
        ],
        out_specs=pl.BlockSpec(memory_space=pltpu.MemorySpace.HBM),
        scratch_shapes=[
            pltpu.VMEM((b, hw2, c2), jnp.float32),
            pltpu.VMEM((2, rows, 2 * cout), jnp.float32),
            pltpu.VMEM((c2, 2 * cout), jnp.float32),
            pltpu.VMEM((N_DEV, 2 * b, c2), jnp.float32),
            pltpu.SemaphoreType.DMA((b * NCHUNK,)),
            pltpu.SemaphoreType.DMA((2,)),
            pltpu.SemaphoreType.DMA((N_DEV - 1,)),
            pltpu.SemaphoreType.DMA((N_DEV - 1,)),
        ],
        compiler_params=pltpu.CompilerParams(collective_id=0),
    )(xp, Wp)
    return out.reshape(b, h, w, cout)


# device time: 36711 ns/iter; 1.2291x vs baseline; 1.2291x over previous
import os

import jax
import jax.numpy as jnp
from jax import lax
from jax.experimental import pallas as pl
from jax.experimental.pallas import tpu as pltpu

N_DEV = 4
NCHUNK = 4
KVAR = os.environ.get("KVAR", "")


def kernel(x, Wp):
    b, h, w, c = x.shape
    cout = Wp.shape[1]
    hw = h * w
    rows = hw // NCHUNK
    n_global = (h * N_DEV) * w
    x3 = x.reshape(b, hw, c)

    def body(
        x_hbm,
        wp_ref,
        out_hbm,
        x_vmem,
        outbuf,
        stats_ref,
        copy_sems,
        out_sems,
        send_sems,
        recv_sems,
    ):
        my = lax.axis_index("i")

        if KVAR != "nocomm":
            barrier_sem = pltpu.get_barrier_semaphore()
            for d in (1, 2, 3):
                pl.semaphore_signal(
                    barrier_sem,
                    inc=1,
                    device_id=((my + d) % N_DEV,),
                    device_id_type=pl.DeviceIdType.MESH,
                )

        copies = []
        for bi in range(b):
            for ci in range(NCHUNK):
                cp = pltpu.make_async_copy(
                    x_hbm.at[bi, pl.ds(ci * rows, rows), :],
                    x_vmem.at[bi, pl.ds(ci * rows, rows), :],
                    copy_sems.at[bi * NCHUNK + ci],
                )
                cp.start()
                copies.append(cp)
        nstat = 1 if KVAR == "nostats" else NCHUNK
        for bi in range(b):
            s = None
            sq = None
            for ci in range(NCHUNK):
                copies[bi * NCHUNK + ci].wait()
                if ci >= nstat:
                    continue
                xb = x_vmem[bi, pl.ds(ci * rows, rows), :]
                ps = jnp.sum(xb, axis=0, keepdims=True)
                psq = jnp.sum(xb * xb, axis=0, keepdims=True)
                s = ps if s is None else s + ps
                sq = psq if sq is None else sq + psq
            stats_ref[N_DEV - 1, 2 * bi : 2 * bi + 1, :] = s
            stats_ref[N_DEV - 1, 2 * bi + 1 : 2 * bi + 2, :] = sq

        if KVAR != "nocomm":
            pl.semaphore_wait(barrier_sem, N_DEV - 1)
            sends = []
            for d in (1, 2, 3):
                rdma = pltpu.make_async_remote_copy(
                    src_ref=stats_ref.at[N_DEV - 1],
                    dst_ref=stats_ref.at[d - 1],
                    send_sem=send_sems.at[d - 1],
                    recv_sem=recv_sems.at[d - 1],
                    device_id=((my + d) % N_DEV,),
                    device_id_type=pl.DeviceIdType.MESH,
                )
                rdma.start()
                sends.append(rdma)
            for d in (1, 2, 3):
                recv = pltpu.make_async_remote_copy(
                    src_ref=stats_ref.at[N_DEV - 1],
                    dst_ref=stats_ref.at[d - 1],
                    send_sem=send_sems.at[d - 1],
                    recv_sem=recv_sems.at[d - 1],
                    device_id=((my + d) % N_DEV,),
                    device_id_type=pl.DeviceIdType.MESH,
                )
                recv.wait_recv()
            for rdma in sends:
                rdma.wait_send()

        eps = 1e-5
        inv_n = 1.0 / float(n_global)
        means = []
        scales = []
        nslot = 1 if KVAR == "nocomm" else N_DEV
        for bi in range(b):
            idx0 = N_DEV - 1 if KVAR == "nocomm" else 0
            ssum = stats_ref[idx0, 2 * bi : 2 * bi + 1, :]
            ssq = stats_ref[idx0, 2 * bi + 1 : 2 * bi + 2, :]
            if nslot == N_DEV:
                for sl in range(1, N_DEV):
                    ssum = ssum + stats_ref[sl, 2 * bi : 2 * bi + 1, :]
                    ssq = ssq + stats_ref[sl, 2 * bi + 1 : 2 * bi + 2, :]
            mean = ssum * inv_n
            var = ssq * inv_n - mean * mean
            means.append(mean)
            scales.append(lax.rsqrt(var + eps))

        wp = wp_ref[:, :]
        out_waits = [None, None]
        k = 0
        for bi in range(b):
            for ci in range(NCHUNK):
                slot = k % 2
                if out_waits[slot] is not None:
                    out_waits[slot].wait()
                xb = x_vmem[bi, pl.ds(ci * rows, rows), :]
                hh = (xb - means[bi]) * scales[bi]
                if KVAR == "nosilu":
                    a = hh
                else:
                    a = hh * jax.nn.sigmoid(hh)
                if KVAR == "nomatmul":
                    outbuf[slot, :, 0:c] = a
                else:
                    outbuf[slot] = jnp.dot(
                        a, wp, preferred_element_type=jnp.float32
                    )
                cp = pltpu.make_async_copy(
                    outbuf.at[slot],
                    out_hbm.at[bi, pl.ds(ci * rows, rows), :],
                    out_sems.at[slot],
                )
                cp.start()
                out_waits[slot] = cp
                k += 1
        out_waits[0].wait()
        out_waits[1].wait()

    out = pl.pallas_call(
        body,
        out_shape=jax.ShapeDtypeStruct((b, hw, cout), jnp.float32),
        in_specs=[
            pl.BlockSpec(memory_space=pltpu.MemorySpace.HBM),
            pl.BlockSpec(memory_space=pltpu.MemorySpace.VMEM),
        ],
        out_specs=pl.BlockSpec(memory_space=pltpu.MemorySpace.HBM),
        scratch_shapes=[
            pltpu.VMEM((b, hw, c), jnp.float32),
            pltpu.VMEM((2, rows, cout), jnp.float32),
            pltpu.VMEM((N_DEV, 2 * b, c), jnp.float32),
            pltpu.SemaphoreType.DMA((b * NCHUNK,)),
            pltpu.SemaphoreType.DMA((2,)),
            pltpu.SemaphoreType.DMA((N_DEV - 1,)),
            pltpu.SemaphoreType.DMA((N_DEV - 1,)),
        ],
        compiler_params=pltpu.CompilerParams(collective_id=0),
    )(x3, Wp)
    return out.reshape(b, h, w, cout)


# device time: 35621 ns/iter; 1.2668x vs baseline; 1.0306x over previous
import os

import jax
import jax.numpy as jnp
from jax import lax
from jax.experimental import pallas as pl
from jax.experimental.pallas import tpu as pltpu

N_DEV = 4
NCHUNK = 4
KVAR = os.environ.get("KVAR", "")


def kernel(x, Wp):
    b, h, w, c = x.shape
    cout = Wp.shape[1]
    hw = h * w
    rows = hw // NCHUNK
    n_global = (h * N_DEV) * w
    x3 = x.reshape(b, hw, c)

    def body(
        x_hbm,
        wp_ref,
        out_hbm,
        x_vmem,
        outbuf,
        stats_ref,
        copy_sems,
        out_sems,
        send_sems,
        recv_sems,
    ):
        my = lax.axis_index("i")

        if KVAR != "nocomm":
            barrier_sem = pltpu.get_barrier_semaphore()
            for d in (1, 2, 3):
                pl.semaphore_signal(
                    barrier_sem,
                    inc=1,
                    device_id=((my + d) % N_DEV,),
                    device_id_type=pl.DeviceIdType.MESH,
                )

        copies = []
        for bi in range(b):
            for ci in range(NCHUNK):
                cp = pltpu.make_async_copy(
                    x_hbm.at[bi, pl.ds(ci * rows, rows), :],
                    x_vmem.at[bi, pl.ds(ci * rows, rows), :],
                    copy_sems.at[bi * NCHUNK + ci],
                )
                cp.start()
                copies.append(cp)
        nstat = 1 if KVAR == "nostats" else NCHUNK
        for bi in range(b):
            s = None
            sq = None
            for ci in range(NCHUNK):
                copies[bi * NCHUNK + ci].wait()
                if ci >= nstat:
                    continue
                xb = x_vmem[bi, pl.ds(ci * rows, rows), :]
                ps = jnp.sum(xb, axis=0, keepdims=True)
                psq = jnp.sum(xb * xb, axis=0, keepdims=True)
                s = ps if s is None else s + ps
                sq = psq if sq is None else sq + psq
            stats_ref[N_DEV - 1, 2 * bi : 2 * bi + 1, :] = s
            stats_ref[N_DEV - 1, 2 * bi + 1 : 2 * bi + 2, :] = sq

        if KVAR != "nocomm":
            pl.semaphore_wait(barrier_sem, N_DEV - 1)
            sends = []
            for d in (1, 2, 3):
                rdma = pltpu.make_async_remote_copy(
                    src_ref=stats_ref.at[N_DEV - 1],
                    dst_ref=stats_ref.at[d - 1],
                    send_sem=send_sems.at[d - 1],
                    recv_sem=recv_sems.at[d - 1],
                    device_id=((my + d) % N_DEV,),
                    device_id_type=pl.DeviceIdType.MESH,
                )
                rdma.start()
                sends.append(rdma)
            for d in (1, 2, 3):
                recv = pltpu.make_async_remote_copy(
                    src_ref=stats_ref.at[N_DEV - 1],
                    dst_ref=stats_ref.at[d - 1],
                    send_sem=send_sems.at[d - 1],
                    recv_sem=recv_sems.at[d - 1],
                    device_id=((my + d) % N_DEV,),
                    device_id_type=pl.DeviceIdType.MESH,
                )
                recv.wait_recv()
            for rdma in sends:
                rdma.wait_send()

        eps = 1e-5
        inv_n = 1.0 / float(n_global)
        means = []
        scales = []
        nslot = 1 if KVAR == "nocomm" else N_DEV
        for bi in range(b):
            idx0 = N_DEV - 1 if KVAR == "nocomm" else 0
            ssum = stats_ref[idx0, 2 * bi : 2 * bi + 1, :]
            ssq = stats_ref[idx0, 2 * bi + 1 : 2 * bi + 2, :]
            if nslot == N_DEV:
                for sl in range(1, N_DEV):
                    ssum = ssum + stats_ref[sl, 2 * bi : 2 * bi + 1, :]
                    ssq = ssq + stats_ref[sl, 2 * bi + 1 : 2 * bi + 2, :]
            mean = ssum * inv_n
            var = ssq * inv_n - mean * mean
            means.append(mean)
            scales.append(lax.rsqrt(var + eps))

        wp = wp_ref[:, :]
        out_waits = [None, None]
        k = 0
        for bi in range(b):
            for ci in range(NCHUNK):
                slot = k % 2
                if out_waits[slot] is not None:
                    out_waits[slot].wait()
                xb = x_vmem[bi, pl.ds(ci * rows, rows), :]
                hh = (xb - means[bi]) * scales[bi]
                if KVAR == "nosilu":
                    a = hh
                else:
                    a = hh * jax.nn.sigmoid(hh)
                if KVAR == "nomatmul":
                    outbuf[slot, :, 0:c] = a
                else:
                    outbuf[slot] = jnp.dot(
                        a, wp, preferred_element_type=jnp.float32
                    )
                cp = pltpu.make_async_copy(
                    outbuf.at[slot],
                    out_hbm.at[bi, pl.ds(ci * rows, rows), :],
                    out_sems.at[slot],
                )
                cp.start()
                out_waits[slot] = cp
                k += 1
        out_waits[0].wait()
        out_waits[1].wait()

    out = pl.pallas_call(
        body,
        out_shape=jax.ShapeDtypeStruct((b, hw, cout), jnp.float32),
        in_specs=[
            pl.BlockSpec(memory_space=pltpu.MemorySpace.HBM),
            pl.BlockSpec(memory_space=pltpu.MemorySpace.VMEM),
        ],
        out_specs=pl.BlockSpec(memory_space=pltpu.MemorySpace.HBM),
        scratch_shapes=[
            pltpu.VMEM((b, hw, c), jnp.float32),
            pltpu.VMEM((2, rows, cout), jnp.float32),
            pltpu.VMEM((N_DEV, 2 * b, c), jnp.float32),
            pltpu.SemaphoreType.DMA((b * NCHUNK,)),
            pltpu.SemaphoreType.DMA((2,)),
            pltpu.SemaphoreType.DMA((N_DEV - 1,)),
            pltpu.SemaphoreType.DMA((N_DEV - 1,)),
        ],
        compiler_params=(
            pltpu.CompilerParams()
            if KVAR == "nocomm"
            else pltpu.CompilerParams(collective_id=0)
        ),
    )(x3, Wp)
    return out.reshape(b, h, w, cout)


# device time: 17149 ns/iter; 2.6312x vs baseline; 2.0771x over previous
import os

import jax
import jax.numpy as jnp
from jax import lax
from jax.experimental import pallas as pl
from jax.experimental.pallas import tpu as pltpu

N_DEV = 4
NCHUNK = 4
KVAR = os.environ.get("KVAR", "")
DO_COMM = KVAR not in ("nocomm", "dmaonly")
DO_INDMA = KVAR != "commonly"
DO_COMPUTE = KVAR not in ("commonly", "dmaonly")
DO_OUT = KVAR != "commonly"


def kernel(x, Wp):
    b, h, w, c = x.shape
    cout = Wp.shape[1]
    hw = h * w
    rows = hw // NCHUNK
    n_global = (h * N_DEV) * w
    x3 = x.reshape(b, hw, c)

    def body(
        x_hbm,
        wp_ref,
        out_hbm,
        x_vmem,
        outbuf,
        stats_ref,
        copy_sems,
        out_sems,
        send_sems,
        recv_sems,
    ):
        my = lax.axis_index("i")

        if DO_COMM:
            barrier_sem = pltpu.get_barrier_semaphore()
            for d in (1, 2, 3):
                pl.semaphore_signal(
                    barrier_sem,
                    inc=1,
                    device_id=((my + d) % N_DEV,),
                    device_id_type=pl.DeviceIdType.MESH,
                )

        if DO_INDMA:
            copies = []
            for bi in range(b):
                for ci in range(NCHUNK):
                    cp = pltpu.make_async_copy(
                        x_hbm.at[bi, pl.ds(ci * rows, rows), :],
                        x_vmem.at[bi, pl.ds(ci * rows, rows), :],
                        copy_sems.at[bi * NCHUNK + ci],
                    )
                    cp.start()
                    copies.append(cp)
        nstat = 1 if KVAR == "nostats" else NCHUNK
        for bi in range(b):
            s = None
            sq = None
            for ci in range(NCHUNK):
                if DO_INDMA:
                    copies[bi * NCHUNK + ci].wait()
                if not DO_COMPUTE or ci >= nstat:
                    continue
                xb = x_vmem[bi, pl.ds(ci * rows, rows), :]
                ps = jnp.sum(xb, axis=0, keepdims=True)
                psq = jnp.sum(xb * xb, axis=0, keepdims=True)
                s = ps if s is None else s + ps
                sq = psq if sq is None else sq + psq
            if s is not None:
                stats_ref[N_DEV - 1, 2 * bi : 2 * bi + 1, :] = s
                stats_ref[N_DEV - 1, 2 * bi + 1 : 2 * bi + 2, :] = sq

        if DO_COMM:
            pl.semaphore_wait(barrier_sem, N_DEV - 1)
            sends = []
            for d in (1, 2, 3):
                rdma = pltpu.make_async_remote_copy(
                    src_ref=stats_ref.at[N_DEV - 1],
                    dst_ref=stats_ref.at[d - 1],
                    send_sem=send_sems.at[d - 1],
                    recv_sem=recv_sems.at[d - 1],
                    device_id=((my + d) % N_DEV,),
                    device_id_type=pl.DeviceIdType.MESH,
                )
                rdma.start()
                sends.append(rdma)
            for d in (1, 2, 3):
                recv = pltpu.make_async_remote_copy(
                    src_ref=stats_ref.at[N_DEV - 1],
                    dst_ref=stats_ref.at[d - 1],
                    send_sem=send_sems.at[d - 1],
                    recv_sem=recv_sems.at[d - 1],
                    device_id=((my + d) % N_DEV,),
                    device_id_type=pl.DeviceIdType.MESH,
                )
                recv.wait_recv()
            for rdma in sends:
                rdma.wait_send()

        eps = 1e-5
        inv_n = 1.0 / float(n_global)
        means = []
        scales = []
        nslot = 1 if not DO_COMM else N_DEV
        for bi in range(b):
            if not DO_COMPUTE:
                continue
            idx0 = N_DEV - 1 if not DO_COMM else 0
            ssum = stats_ref[idx0, 2 * bi : 2 * bi + 1, :]
            ssq = stats_ref[idx0, 2 * bi + 1 : 2 * bi + 2, :]
            if nslot == N_DEV:
                for sl in range(1, N_DEV):
                    ssum = ssum + stats_ref[sl, 2 * bi : 2 * bi + 1, :]
                    ssq = ssq + stats_ref[sl, 2 * bi + 1 : 2 * bi + 2, :]
            mean = ssum * inv_n
            var = ssq * inv_n - mean * mean
            means.append(mean)
            scales.append(lax.rsqrt(var + eps))

        wp = wp_ref[:, :]
        out_waits = [None, None]
        k = 0
        for bi in range(b):
            for ci in range(NCHUNK):
                if not DO_OUT:
                    continue
                slot = k % 2
                if out_waits[slot] is not None:
                    out_waits[slot].wait()
                if DO_COMPUTE:
                    xb = x_vmem[bi, pl.ds(ci * rows, rows), :]
                    hh = (xb - means[bi]) * scales[bi]
                    if KVAR == "nosilu":
                        a = hh
                    else:
                        a = hh * jax.nn.sigmoid(hh)
                    if KVAR == "nomatmul":
                        outbuf[slot, :, 0:c] = a
                    else:
                        outbuf[slot] = jnp.dot(
                            a, wp, preferred_element_type=jnp.float32
                        )
                cp = pltpu.make_async_copy(
                    outbuf.at[slot],
                    out_hbm.at[bi, pl.ds(ci * rows, rows), :],
                    out_sems.at[slot],
                )
                cp.start()
                out_waits[slot] = cp
                k += 1
        if out_waits[0] is not None:
            out_waits[0].wait()
        if out_waits[1] is not None:
            out_waits[1].wait()

    out = pl.pallas_call(
        body,
        out_shape=jax.ShapeDtypeStruct((b, hw, cout), jnp.float32),
        in_specs=[
            pl.BlockSpec(memory_space=pltpu.MemorySpace.HBM),
            pl.BlockSpec(memory_space=pltpu.MemorySpace.VMEM),
        ],
        out_specs=pl.BlockSpec(memory_space=pltpu.MemorySpace.HBM),
        scratch_shapes=[
            pltpu.VMEM((b, hw, c), jnp.float32),
            pltpu.VMEM((2, rows, cout), jnp.float32),
            pltpu.VMEM((N_DEV, 2 * b, c), jnp.float32),
            pltpu.SemaphoreType.DMA((b * NCHUNK,)),
            pltpu.SemaphoreType.DMA((2,)),
            pltpu.SemaphoreType.DMA((N_DEV - 1,)),
            pltpu.SemaphoreType.DMA((N_DEV - 1,)),
        ],
        compiler_params=(
            pltpu.CompilerParams(collective_id=0)
            if DO_COMM
            else pltpu.CompilerParams()
        ),
    )(x3, Wp)
    return out.reshape(b, h, w, cout)
